# baseline (speedup 1.0000x reference)
import dataclasses
import functools

import jax
import jax.numpy as jnp
from jax import lax
from jax.experimental import pallas as pl
from jax.experimental.pallas import tpu as pltpu
from jax.experimental.pallas import tpu_sc as plsc


N_EMB = 5
N_INT = 5
D_PACK = 48


def _sc_gather_rows(table, idx, n_idx_padded):
    n_cores, n_sub = 2, 16
    n_workers = n_cores * n_sub
    b_per_w = n_idx_padded // n_workers
    mesh = plsc.VectorSubcoreMesh(core_axis_name="c", subcore_axis_name="s")
    cp = pltpu.CompilerParams()
    fields = pltpu.CompilerParams.__dataclass_fields__
    if "use_tc_tiling_on_sc" in fields:
        cp = dataclasses.replace(cp, use_tc_tiling_on_sc=False)

    @functools.partial(
        pl.kernel,
        mesh=mesh,
        compiler_params=cp,
        out_type=jax.ShapeDtypeStruct((n_idx_padded, D_PACK), table.dtype),
        scratch_types=[
            pltpu.VMEM((b_per_w,), jnp.int32),
            pltpu.VMEM((b_per_w, D_PACK), table.dtype),
            pltpu.SemaphoreType.DMA,
        ],
    )
    def gather_kernel(table_hbm, idx_hbm, out_hbm, idx_v, rows_v, sem):
        wid = lax.axis_index("s") * n_cores + lax.axis_index("c")
        base = wid * b_per_w
        pltpu.sync_copy(idx_hbm.at[pl.ds(base, b_per_w)], idx_v)
        pltpu.async_copy(table_hbm.at[idx_v], rows_v, sem).wait()
        pltpu.sync_copy(rows_v, out_hbm.at[pl.ds(base, b_per_w)])

    return gather_kernel(table, idx)


def _dense_body(xt_ref, p_ref, out_ref):
    x = [xt_ref[i] for i in range(N_EMB)]
    out = None
    for j in range(N_INT):
        h = p_ref[25 + j : 26 + j, :]
        for i in range(N_EMB):
            h = h + x[i] * p_ref[i * N_INT + j : i * N_INT + j + 1, :]
        h = jax.nn.sigmoid(h)
        t = h * p_ref[30 + j : 31 + j, :]
        out = t if out is None else out + t
    out_ref[...] = out + p_ref[35:36, :]


def _dense_mlp(xt, params):
    _, C, G = xt.shape
    CB = 512
    return pl.pallas_call(
        _dense_body,
        grid=(C // CB,),
        in_specs=[
            pl.BlockSpec((N_EMB, CB, G), lambda c: (0, c, 0)),
            pl.BlockSpec((D_PACK, G), lambda c: (0, 0)),
        ],
        out_specs=pl.BlockSpec((CB, G), lambda c: (c, 0)),
        out_shape=jax.ShapeDtypeStruct((C, G), jnp.float32),
    )(xt, params)


def kernel(cell_gene_embedding, gene_ix, W1, b1, W2, b2):
    N = W1.shape[0]
    G = gene_ix.shape[0]
    Gp = 1024

    table = jnp.concatenate(
        [
            W1.reshape(N, N_EMB * N_INT),
            b1,
            W2.reshape(N, N_INT),
            b2,
            jnp.zeros((N, D_PACK - 36), jnp.float32),
        ],
        axis=1,
    )
    idx = jnp.pad(gene_ix.astype(jnp.int32), (0, Gp - G))
    rows = _sc_gather_rows(table, idx, Gp)
    params = rows[:G].T
    xt = jnp.transpose(cell_gene_embedding, (2, 0, 1))
    return _dense_mlp(xt, params)
